# baseline (speedup 1.0000x reference)
import functools

import jax
import jax.numpy as jnp
from jax import lax
from jax.experimental import pallas as pl
from jax.experimental.pallas import tpu as pltpu
from jax.experimental.pallas import tpu_sc as plsc

N = 10000
E = 160000
DF = 128
H = 32
HP = 128
GH = 64
NG = 128
HH = H * H

NC, NS = 2, 16
NW = NC * NS
CHUNK = 128
NJ = E // CHUNK
T = -(-NJ // NW)
EPAD = NW * T * CHUNK


def _sc_mesh():
    return plsc.VectorSubcoreMesh(core_axis_name="c", subcore_axis_name="s",
                                  num_cores=NC, num_subcores=NS)


def _sc_gather(table, idx3):
    @functools.partial(
        pl.kernel,
        out_type=jax.ShapeDtypeStruct((E, HP), jnp.float32),
        mesh=_sc_mesh(),
        scratch_types=[
            pltpu.VMEM((T, CHUNK), jnp.int32),
            pltpu.VMEM((2, CHUNK, HP), jnp.float32),
            pltpu.SemaphoreType.DMA((2,)),
        ],
    )
    def k(table_hbm, idx_hbm, out_hbm, idx_v, rows_v, sem):
        wid = lax.axis_index("s") * NC + lax.axis_index("c")
        nc = jnp.maximum(0, jnp.minimum(T, NJ - wid * T))
        pltpu.sync_copy(idx_hbm.at[wid], idx_v)

        @pl.when(nc > 0)
        def _():
            pltpu.async_copy(table_hbm.at[idx_v.at[0]], rows_v.at[0], sem.at[0])

        def body(t, carry):
            p = lax.rem(t, 2)
            pn = lax.rem(t + 1, 2)

            @pl.when(t + 1 < nc)
            def _():
                pltpu.async_copy(table_hbm.at[idx_v.at[t + 1]],
                                 rows_v.at[pn], sem.at[pn])

            pltpu.make_async_copy(table_hbm.at[idx_v.at[t]],
                                  rows_v.at[p], sem.at[p]).wait()
            pltpu.sync_copy(rows_v.at[p],
                            out_hbm.at[pl.ds((wid * T + t) * CHUNK, CHUNK)])
            return carry

        lax.fori_loop(0, nc, body, 0)

    return k(table, idx3)


def _sc_scatter_add(msg, dst3, zeros):
    wb = N // 10

    @functools.partial(
        pl.kernel,
        out_type=jax.ShapeDtypeStruct((NC, N, HP), jnp.float32),
        mesh=_sc_mesh(),
        scratch_types=[
            pltpu.VMEM((T, CHUNK), jnp.int32),
            pltpu.VMEM((2, CHUNK, HP), jnp.float32),
            pltpu.VMEM_SHARED((N, HP), jnp.float32),
            pltpu.SemaphoreType.DMA((2,)),
        ],
    )
    def k(msg_hbm, dst_hbm, zeros_hbm, out_hbm, dst_v, rows_v, acc_sh, sem):
        cid = lax.axis_index("c")
        sid = lax.axis_index("s")
        wid = sid * NC + cid
        nc = jnp.maximum(0, jnp.minimum(T, NJ - wid * T))

        @pl.when(sid == 0)
        def _():
            pltpu.sync_copy(zeros_hbm, acc_sh)

        plsc.subcore_barrier()
        pltpu.sync_copy(dst_hbm.at[wid], dst_v)

        @pl.when(nc > 0)
        def _():
            pltpu.async_copy(msg_hbm.at[pl.ds(wid * T * CHUNK, CHUNK)],
                             rows_v.at[0], sem.at[0])

        def body(t, carry):
            p = lax.rem(t, 2)
            pn = lax.rem(t + 1, 2)

            @pl.when(t + 1 < nc)
            def _():
                pltpu.async_copy(
                    msg_hbm.at[pl.ds((wid * T + t + 1) * CHUNK, CHUNK)],
                    rows_v.at[pn], sem.at[pn])

            pltpu.make_async_copy(
                msg_hbm.at[pl.ds((wid * T + t) * CHUNK, CHUNK)],
                rows_v.at[p], sem.at[p]).wait()
            pltpu.sync_copy(rows_v.at[p], acc_sh.at[dst_v.at[t]], add=True)
            return carry

        lax.fori_loop(0, nc, body, 0)
        plsc.subcore_barrier()

        @pl.when(sid < 10)
        def _():
            pltpu.sync_copy(acc_sh.at[pl.ds(sid * wb, wb)],
                            out_hbm.at[cid].at[pl.ds(sid * wb, wb)])

    return k(msg, dst3, zeros)


_BE = 1280
_BN = 2000


def _tc_pre_mlp(X, p0W, p0b, p1W, p1b, p2W, p2b):
    def body(x_ref, w0, b0, w1, b1, w2, b2, o_ref):
        o = jnp.maximum(jnp.dot(x_ref[...], w0[...],
                                preferred_element_type=jnp.float32) + b0[...], 0.0)
        o = jnp.maximum(jnp.dot(o, w1[...],
                                preferred_element_type=jnp.float32) + b1[...], 0.0)
        o = jnp.maximum(jnp.dot(o, w2[...],
                                preferred_element_type=jnp.float32) + b2[...], 0.0)
        o_ref[...] = jnp.concatenate(
            [o, jnp.zeros((_BN, HP - H), jnp.float32)], axis=1)

    full = lambda shape: pl.BlockSpec(shape, lambda i: (0, 0))
    return pl.pallas_call(
        body,
        grid=(N // _BN,),
        in_specs=[pl.BlockSpec((_BN, DF), lambda i: (i, 0)),
                  full((DF, H)), full((1, H)),
                  full((H, H)), full((1, H)),
                  full((H, H)), full((1, H))],
        out_specs=pl.BlockSpec((_BN, HP), lambda i: (i, 0)),
        out_shape=jax.ShapeDtypeStruct((N, HP), jnp.float32),
    )(X, p0W, p0b, p1W, p1b, p2W, p2b)


def _tc_messages(ea, y, w0, b0, W1a, Rt):
    def body(ea_ref, y_ref, w0r, b0r, W1r, Rtr, o_ref):
        u1 = jnp.maximum(ea_ref[...] * w0r[...] + b0r[...], 0.0)
        wf = jnp.dot(u1.astype(jnp.bfloat16), W1r[...],
                     preferred_element_type=jnp.float32)
        yt = jnp.tile(y_ref[...][:, :H], (1, H))
        P = (wf * yt).astype(jnp.bfloat16)
        msg = jnp.dot(P, Rtr[...], preferred_element_type=jnp.float32)
        o_ref[:, :H] = msg
        o_ref[:, H:2 * H] = jnp.ones((_BE, H), jnp.float32)

    full = lambda shape: pl.BlockSpec(shape, lambda i: (0, 0))
    return pl.pallas_call(
        body,
        grid=(E // _BE,),
        in_specs=[pl.BlockSpec((_BE, 1), lambda i: (i, 0)),
                  pl.BlockSpec((_BE, HP), lambda i: (i, 0)),
                  full((1, GH + 1)), full((1, GH + 1)),
                  full((GH + 1, HH)), full((HH, H))],
        out_specs=pl.BlockSpec((_BE, HP), lambda i: (i, 0)),
        out_shape=jax.ShapeDtypeStruct((E, HP), jnp.float32),
    )(ea, y, w0, b0, W1a, Rt)


def _tc_gru(h, agg0, agg1, Wroot, broot, WihT, bih, WhhT, bhh):
    def body(h_ref, a0, a1, wr, br, wih, bi, whh, bh, o_ref):
        hv = h_ref[...][:, :H]
        a0v = a0[...]
        a1v = a1[...]
        cnt = a0v[:, H:H + 1] + a1v[:, H:H + 1]
        inv = 1.0 / jnp.maximum(cnt, 1.0)
        agg = (a0v[:, :H] + a1v[:, :H]) * inv
        conv = jnp.dot(hv, wr[...],
                       preferred_element_type=jnp.float32) + br[...] + agg
        gi = jnp.dot(conv, wih[...],
                     preferred_element_type=jnp.float32) + bi[...]
        gh = jnp.dot(hv, whh[...],
                     preferred_element_type=jnp.float32) + bh[...]
        r = jax.nn.sigmoid(gi[:, 0:H] + gh[:, 0:H])
        z = jax.nn.sigmoid(gi[:, H:2 * H] + gh[:, H:2 * H])
        n = jnp.tanh(gi[:, 2 * H:] + r * gh[:, 2 * H:])
        o = (1.0 - z) * n + z * hv
        o_ref[...] = jnp.concatenate(
            [o, jnp.zeros((_BN, HP - H), jnp.float32)], axis=1)

    full = lambda shape: pl.BlockSpec(shape, lambda i: (0, 0))
    blk = lambda w: pl.BlockSpec((_BN, w), lambda i: (i, 0))
    return pl.pallas_call(
        body,
        grid=(N // _BN,),
        in_specs=[blk(HP), blk(HP), blk(HP),
                  full((H, H)), full((1, H)),
                  full((H, 3 * H)), full((1, 3 * H)),
                  full((H, 3 * H)), full((1, 3 * H))],
        out_specs=blk(HP),
        out_shape=jax.ShapeDtypeStruct((N, HP), jnp.float32),
    )(h, agg0, agg1, Wroot, broot, WihT, bih, WhhT, bhh)


def _tc_pool_post(x, bm, q0W, q0b, q1W, q1b, q2W, q2b, q3W, q3b):
    G = N // _BN

    def body(x_ref, bm_ref, w0, b0, w1, b1, w2, b2, w3, b3, o_ref, acc, cnt):
        i = pl.program_id(0)

        @pl.when(i == 0)
        def _():
            acc[...] = jnp.zeros_like(acc)
            cnt[...] = jnp.zeros_like(cnt)

        ids = bm_ref[...]
        onehot = (ids == lax.broadcasted_iota(jnp.int32, (_BN, NG), 1)
                  ).astype(jnp.float32)
        dn = (((0,), (0,)), ((), ()))
        acc[...] += lax.dot_general(onehot, x_ref[...][:, :H], dn,
                                    preferred_element_type=jnp.float32)
        cnt[...] += lax.dot_general(onehot, jnp.ones((_BN, H), jnp.float32), dn,
                                    preferred_element_type=jnp.float32)

        @pl.when(i == G - 1)
        def _():
            m = acc[...] / jnp.maximum(cnt[...], 1.0)
            o = jnp.maximum(jnp.dot(m, w0[...],
                                    preferred_element_type=jnp.float32) + b0[...], 0.0)
            o = jnp.maximum(jnp.dot(o, w1[...],
                                    preferred_element_type=jnp.float32) + b1[...], 0.0)
            o = jnp.maximum(jnp.dot(o, w2[...],
                                    preferred_element_type=jnp.float32) + b2[...], 0.0)
            o_ref[...] = jnp.dot(o, w3[...],
                                 preferred_element_type=jnp.float32) + b3[...]

    full = lambda shape: pl.BlockSpec(shape, lambda i: (0, 0))
    return pl.pallas_call(
        body,
        grid=(G,),
        in_specs=[pl.BlockSpec((_BN, HP), lambda i: (i, 0)),
                  pl.BlockSpec((_BN, 1), lambda i: (i, 0)),
                  full((H, H)), full((1, H)),
                  full((H, H)), full((1, H)),
                  full((H, H)), full((1, H)),
                  full((H, 1)), full((1, 1))],
        out_specs=pl.BlockSpec((NG, 1), lambda i: (0, 0)),
        out_shape=jax.ShapeDtypeStruct((NG, 1), jnp.float32),
        scratch_shapes=[pltpu.VMEM((NG, H), jnp.float32),
                        pltpu.VMEM((NG, H), jnp.float32)],
    )(x, bm, q0W, q0b, q1W, q1b, q2W, q2b, q3W, q3b)


def kernel(X, edge_idx, edge_attr, batch_map, params):
    pad = jnp.zeros((EPAD - E,), jnp.int32)
    src2 = jnp.concatenate([edge_idx[0], pad]).reshape(NW, T, CHUNK)
    dst2 = jnp.concatenate([edge_idx[1], pad]).reshape(NW, T, CHUNK)
    ea = edge_attr.reshape(E, 1)
    bm = batch_map.reshape(N, 1)
    zeros = jnp.zeros((N, HP), jnp.float32)

    row = lambda b: b.reshape(1, -1)
    Rt = jnp.kron(jnp.eye(H, dtype=jnp.bfloat16),
                  jnp.ones((H, 1), jnp.bfloat16))

    pre = params['pre']
    out = _tc_pre_mlp(X, pre[0]['W'], row(pre[0]['b']),
                      pre[1]['W'], row(pre[1]['b']),
                      pre[2]['W'], row(pre[2]['b']))

    for g in params['gcn']:
        W1 = g['edge_nn1']['W']
        b1 = g['edge_nn1']['b']
        W1p = W1.reshape(GH, H, H).transpose(0, 2, 1).reshape(GH, HH)
        b1p = b1.reshape(H, H).T.reshape(1, HH)
        W1a = jnp.concatenate([W1p, b1p], axis=0).astype(jnp.bfloat16)

        w0a = jnp.concatenate([g['edge_nn0']['W'].reshape(1, GH),
                               jnp.zeros((1, 1), jnp.float32)], axis=1)
        b0a = jnp.concatenate([g['edge_nn0']['b'].reshape(1, GH),
                               jnp.ones((1, 1), jnp.float32)], axis=1)

        y = _sc_gather(out, src2)
        msg = _tc_messages(ea, y, w0a, b0a, W1a, Rt)
        aggp = _sc_scatter_add(msg, dst2, zeros)
        out = _tc_gru(out, aggp[0], aggp[1],
                      g['root']['W'], row(g['root']['b']),
                      g['gru']['W_ih'].T, row(g['gru']['b_ih']),
                      g['gru']['W_hh'].T, row(g['gru']['b_hh']))

    post = params['post']
    return _tc_pool_post(out, bm,
                         post[0]['W'], row(post[0]['b']),
                         post[1]['W'], row(post[1]['b']),
                         post[2]['W'], row(post[2]['b']),
                         post[3]['W'], row(post[3]['b']))


# speedup vs baseline: 1.0500x; 1.0500x over previous
import functools

import jax
import jax.numpy as jnp
from jax import lax
from jax.experimental import pallas as pl
from jax.experimental.pallas import tpu as pltpu
from jax.experimental.pallas import tpu_sc as plsc

N = 10000
E = 160000
DF = 128
H = 32
HP = 128
GH = 64
NG = 128
HH = H * H

NC, NS = 2, 16
NW = NC * NS
CHUNK = 128
NJ = E // CHUNK
T = -(-NJ // NW)
EPAD = NW * T * CHUNK


def _sc_mesh():
    return plsc.VectorSubcoreMesh(core_axis_name="c", subcore_axis_name="s",
                                  num_cores=NC, num_subcores=NS)


def _sc_gather(table, idx3):
    @functools.partial(
        pl.kernel,
        out_type=jax.ShapeDtypeStruct((E, HP), jnp.float32),
        mesh=_sc_mesh(),
        scratch_types=[
            pltpu.VMEM((T, CHUNK), jnp.int32),
            pltpu.VMEM((2, CHUNK, HP), jnp.float32),
            pltpu.SemaphoreType.DMA((2,)),
        ],
    )
    def k(table_hbm, idx_hbm, out_hbm, idx_v, rows_v, sem):
        wid = lax.axis_index("s") * NC + lax.axis_index("c")
        nc = jnp.maximum(0, jnp.minimum(T, NJ - wid * T))
        pltpu.sync_copy(idx_hbm.at[wid], idx_v)

        @pl.when(nc > 0)
        def _():
            pltpu.async_copy(table_hbm.at[idx_v.at[0]], rows_v.at[0], sem.at[0])

        def body(t, carry):
            p = lax.rem(t, 2)
            pn = lax.rem(t + 1, 2)

            @pl.when(t + 1 < nc)
            def _():
                pltpu.async_copy(table_hbm.at[idx_v.at[t + 1]],
                                 rows_v.at[pn], sem.at[pn])

            pltpu.make_async_copy(table_hbm.at[idx_v.at[t]],
                                  rows_v.at[p], sem.at[p]).wait()
            pltpu.sync_copy(rows_v.at[p],
                            out_hbm.at[pl.ds((wid * T + t) * CHUNK, CHUNK)])
            return carry

        lax.fori_loop(0, nc, body, 0)

    return k(table, idx3)


def _sc_scatter_add(msg, dst3, zeros):
    wb = N // 10

    @functools.partial(
        pl.kernel,
        out_type=jax.ShapeDtypeStruct((NC, N, HP), jnp.float32),
        mesh=_sc_mesh(),
        scratch_types=[
            pltpu.VMEM((T, CHUNK), jnp.int32),
            pltpu.VMEM((2, CHUNK, HP), jnp.float32),
            pltpu.VMEM_SHARED((N, HP), jnp.float32),
            pltpu.SemaphoreType.DMA((2,)),
        ],
    )
    def k(msg_hbm, dst_hbm, zeros_hbm, out_hbm, dst_v, rows_v, acc_sh, sem):
        cid = lax.axis_index("c")
        sid = lax.axis_index("s")
        wid = sid * NC + cid
        nc = jnp.maximum(0, jnp.minimum(T, NJ - wid * T))

        @pl.when(sid == 0)
        def _():
            pltpu.sync_copy(zeros_hbm, acc_sh)

        plsc.subcore_barrier()
        pltpu.sync_copy(dst_hbm.at[wid], dst_v)

        @pl.when(nc > 0)
        def _():
            pltpu.async_copy(msg_hbm.at[pl.ds(wid * T * CHUNK, CHUNK)],
                             rows_v.at[0], sem.at[0])

        def body(t, carry):
            p = lax.rem(t, 2)
            pn = lax.rem(t + 1, 2)

            @pl.when(t + 1 < nc)
            def _():
                pltpu.async_copy(
                    msg_hbm.at[pl.ds((wid * T + t + 1) * CHUNK, CHUNK)],
                    rows_v.at[pn], sem.at[pn])

            pltpu.make_async_copy(
                msg_hbm.at[pl.ds((wid * T + t) * CHUNK, CHUNK)],
                rows_v.at[p], sem.at[p]).wait()
            pltpu.sync_copy(rows_v.at[p], acc_sh.at[dst_v.at[t]], add=True)
            return carry

        lax.fori_loop(0, nc, body, 0)
        plsc.subcore_barrier()

        @pl.when(sid < 10)
        def _():
            pltpu.sync_copy(acc_sh.at[pl.ds(sid * wb, wb)],
                            out_hbm.at[cid].at[pl.ds(sid * wb, wb)])

    return k(msg, dst3, zeros)


_BE = 2000
_BN = 2000


def _tc_pre_mlp(X, p0W, p0b, p1W, p1b, p2W, p2b):
    def body(x_ref, w0, b0, w1, b1, w2, b2, o_ref):
        o = jnp.maximum(jnp.dot(x_ref[...], w0[...],
                                preferred_element_type=jnp.float32) + b0[...], 0.0)
        o = jnp.maximum(jnp.dot(o, w1[...],
                                preferred_element_type=jnp.float32) + b1[...], 0.0)
        o = jnp.maximum(jnp.dot(o, w2[...],
                                preferred_element_type=jnp.float32) + b2[...], 0.0)
        o_ref[...] = jnp.concatenate(
            [o, jnp.zeros((_BN, HP - H), jnp.float32)], axis=1)

    full = lambda shape: pl.BlockSpec(shape, lambda i: (0, 0))
    return pl.pallas_call(
        body,
        grid=(N // _BN,),
        in_specs=[pl.BlockSpec((_BN, DF), lambda i: (i, 0)),
                  full((DF, H)), full((1, H)),
                  full((H, H)), full((1, H)),
                  full((H, H)), full((1, H))],
        out_specs=pl.BlockSpec((_BN, HP), lambda i: (i, 0)),
        out_shape=jax.ShapeDtypeStruct((N, HP), jnp.float32),
    )(X, p0W, p0b, p1W, p1b, p2W, p2b)


def _tc_messages(ea, y, w0, b0, W1a, Rt):
    def body(ea_ref, y_ref, w0r, b0r, W1r, Rtr, o_ref):
        u1 = jnp.maximum(ea_ref[...] * w0r[...] + b0r[...], 0.0)
        wf = jnp.dot(u1.astype(jnp.bfloat16), W1r[...],
                     preferred_element_type=jnp.float32)
        yt = jnp.tile(y_ref[...][:, :H], (1, H))
        P = (wf * yt).astype(jnp.bfloat16)
        msg = jnp.dot(P, Rtr[...], preferred_element_type=jnp.float32)
        o_ref[:, :H] = msg
        o_ref[:, H:2 * H] = jnp.ones((_BE, H), jnp.float32)

    full = lambda shape: pl.BlockSpec(shape, lambda i: (0, 0))
    return pl.pallas_call(
        body,
        grid=(E // _BE,),
        in_specs=[pl.BlockSpec((_BE, 1), lambda i: (i, 0)),
                  pl.BlockSpec((_BE, HP), lambda i: (i, 0)),
                  full((1, GH + 1)), full((1, GH + 1)),
                  full((GH + 1, HH)), full((HH, H))],
        out_specs=pl.BlockSpec((_BE, HP), lambda i: (i, 0)),
        out_shape=jax.ShapeDtypeStruct((E, HP), jnp.float32),
    )(ea, y, w0, b0, W1a, Rt)


def _tc_gru(h, agg0, agg1, Wroot, broot, WihT, bih, WhhT, bhh):
    def body(h_ref, a0, a1, wr, br, wih, bi, whh, bh, o_ref):
        hv = h_ref[...][:, :H]
        a0v = a0[...]
        a1v = a1[...]
        cnt = a0v[:, H:H + 1] + a1v[:, H:H + 1]
        inv = 1.0 / jnp.maximum(cnt, 1.0)
        agg = (a0v[:, :H] + a1v[:, :H]) * inv
        conv = jnp.dot(hv, wr[...],
                       preferred_element_type=jnp.float32) + br[...] + agg
        gi = jnp.dot(conv, wih[...],
                     preferred_element_type=jnp.float32) + bi[...]
        gh = jnp.dot(hv, whh[...],
                     preferred_element_type=jnp.float32) + bh[...]
        r = jax.nn.sigmoid(gi[:, 0:H] + gh[:, 0:H])
        z = jax.nn.sigmoid(gi[:, H:2 * H] + gh[:, H:2 * H])
        n = jnp.tanh(gi[:, 2 * H:] + r * gh[:, 2 * H:])
        o = (1.0 - z) * n + z * hv
        o_ref[...] = jnp.concatenate(
            [o, jnp.zeros((_BN, HP - H), jnp.float32)], axis=1)

    full = lambda shape: pl.BlockSpec(shape, lambda i: (0, 0))
    blk = lambda w: pl.BlockSpec((_BN, w), lambda i: (i, 0))
    return pl.pallas_call(
        body,
        grid=(N // _BN,),
        in_specs=[blk(HP), blk(HP), blk(HP),
                  full((H, H)), full((1, H)),
                  full((H, 3 * H)), full((1, 3 * H)),
                  full((H, 3 * H)), full((1, 3 * H))],
        out_specs=blk(HP),
        out_shape=jax.ShapeDtypeStruct((N, HP), jnp.float32),
    )(h, agg0, agg1, Wroot, broot, WihT, bih, WhhT, bhh)


def _tc_pool_post(x, bm, q0W, q0b, q1W, q1b, q2W, q2b, q3W, q3b):
    G = N // _BN

    def body(x_ref, bm_ref, w0, b0, w1, b1, w2, b2, w3, b3, o_ref, acc, cnt):
        i = pl.program_id(0)

        @pl.when(i == 0)
        def _():
            acc[...] = jnp.zeros_like(acc)
            cnt[...] = jnp.zeros_like(cnt)

        ids = bm_ref[...]
        onehot = (ids == lax.broadcasted_iota(jnp.int32, (_BN, NG), 1)
                  ).astype(jnp.float32)
        dn = (((0,), (0,)), ((), ()))
        acc[...] += lax.dot_general(onehot, x_ref[...][:, :H], dn,
                                    preferred_element_type=jnp.float32)
        cnt[...] += lax.dot_general(onehot, jnp.ones((_BN, H), jnp.float32), dn,
                                    preferred_element_type=jnp.float32)

        @pl.when(i == G - 1)
        def _():
            m = acc[...] / jnp.maximum(cnt[...], 1.0)
            o = jnp.maximum(jnp.dot(m, w0[...],
                                    preferred_element_type=jnp.float32) + b0[...], 0.0)
            o = jnp.maximum(jnp.dot(o, w1[...],
                                    preferred_element_type=jnp.float32) + b1[...], 0.0)
            o = jnp.maximum(jnp.dot(o, w2[...],
                                    preferred_element_type=jnp.float32) + b2[...], 0.0)
            o_ref[...] = jnp.dot(o, w3[...],
                                 preferred_element_type=jnp.float32) + b3[...]

    full = lambda shape: pl.BlockSpec(shape, lambda i: (0, 0))
    return pl.pallas_call(
        body,
        grid=(G,),
        in_specs=[pl.BlockSpec((_BN, HP), lambda i: (i, 0)),
                  pl.BlockSpec((_BN, 1), lambda i: (i, 0)),
                  full((H, H)), full((1, H)),
                  full((H, H)), full((1, H)),
                  full((H, H)), full((1, H)),
                  full((H, 1)), full((1, 1))],
        out_specs=pl.BlockSpec((NG, 1), lambda i: (0, 0)),
        out_shape=jax.ShapeDtypeStruct((NG, 1), jnp.float32),
        scratch_shapes=[pltpu.VMEM((NG, H), jnp.float32),
                        pltpu.VMEM((NG, H), jnp.float32)],
    )(x, bm, q0W, q0b, q1W, q1b, q2W, q2b, q3W, q3b)


def kernel(X, edge_idx, edge_attr, batch_map, params):
    pad = jnp.zeros((EPAD - E,), jnp.int32)
    src2 = jnp.concatenate([edge_idx[0], pad]).reshape(NW, T, CHUNK)
    dst2 = jnp.concatenate([edge_idx[1], pad]).reshape(NW, T, CHUNK)
    ea = edge_attr.reshape(E, 1)
    bm = batch_map.reshape(N, 1)
    zeros = jnp.zeros((N, HP), jnp.float32)

    row = lambda b: b.reshape(1, -1)
    Rt = jnp.kron(jnp.eye(H, dtype=jnp.bfloat16),
                  jnp.ones((H, 1), jnp.bfloat16))

    pre = params['pre']
    out = _tc_pre_mlp(X, pre[0]['W'], row(pre[0]['b']),
                      pre[1]['W'], row(pre[1]['b']),
                      pre[2]['W'], row(pre[2]['b']))

    for g in params['gcn']:
        W1 = g['edge_nn1']['W']
        b1 = g['edge_nn1']['b']
        W1p = W1.reshape(GH, H, H).transpose(0, 2, 1).reshape(GH, HH)
        b1p = b1.reshape(H, H).T.reshape(1, HH)
        W1a = jnp.concatenate([W1p, b1p], axis=0).astype(jnp.bfloat16)

        w0a = jnp.concatenate([g['edge_nn0']['W'].reshape(1, GH),
                               jnp.zeros((1, 1), jnp.float32)], axis=1)
        b0a = jnp.concatenate([g['edge_nn0']['b'].reshape(1, GH),
                               jnp.ones((1, 1), jnp.float32)], axis=1)

        y = _sc_gather(out, src2)
        msg = _tc_messages(ea, y, w0a, b0a, W1a, Rt)
        aggp = _sc_scatter_add(msg, dst2, zeros)
        out = _tc_gru(out, aggp[0], aggp[1],
                      g['root']['W'], row(g['root']['b']),
                      g['gru']['W_ih'].T, row(g['gru']['b_ih']),
                      g['gru']['W_hh'].T, row(g['gru']['b_hh']))

    post = params['post']
    return _tc_pool_post(out, bm,
                         post[0]['W'], row(post[0]['b']),
                         post[1]['W'], row(post[1]['b']),
                         post[2]['W'], row(post[2]['b']),
                         post[3]['W'], row(post[3]['b']))


# speedup vs baseline: 1.0826x; 1.0310x over previous
import functools

import jax
import jax.numpy as jnp
from jax import lax
from jax.experimental import pallas as pl
from jax.experimental.pallas import tpu as pltpu
from jax.experimental.pallas import tpu_sc as plsc

N = 10000
E = 160000
DF = 128
H = 32
HP = 128
GH = 64
NG = 128
HH = H * H

NC, NS = 2, 16
NW = NC * NS
CHUNK = 128
NJ = E // CHUNK
T = -(-NJ // NW)
EPAD = NW * T * CHUNK


def _sc_mesh():
    return plsc.VectorSubcoreMesh(core_axis_name="c", subcore_axis_name="s",
                                  num_cores=NC, num_subcores=NS)


def _sc_gather(table, idx3):
    @functools.partial(
        pl.kernel,
        out_type=jax.ShapeDtypeStruct((E, HP), jnp.float32),
        mesh=_sc_mesh(),
        scratch_types=[
            pltpu.VMEM((T, CHUNK), jnp.int32),
            pltpu.VMEM((2, CHUNK, HP), jnp.float32),
            pltpu.SemaphoreType.DMA((2,)),
        ],
    )
    def k(table_hbm, idx_hbm, out_hbm, idx_v, rows_v, sem):
        wid = lax.axis_index("s") * NC + lax.axis_index("c")
        nc = jnp.maximum(0, jnp.minimum(T, NJ - wid * T))
        pltpu.sync_copy(idx_hbm.at[wid], idx_v)

        @pl.when(nc > 0)
        def _():
            pltpu.async_copy(table_hbm.at[idx_v.at[0]], rows_v.at[0], sem.at[0])

        def body(t, carry):
            p = lax.rem(t, 2)
            pn = lax.rem(t + 1, 2)

            @pl.when(t + 1 < nc)
            def _():
                pltpu.async_copy(table_hbm.at[idx_v.at[t + 1]],
                                 rows_v.at[pn], sem.at[pn])

            pltpu.make_async_copy(table_hbm.at[idx_v.at[t]],
                                  rows_v.at[p], sem.at[p]).wait()
            pltpu.sync_copy(rows_v.at[p],
                            out_hbm.at[pl.ds((wid * T + t) * CHUNK, CHUNK)])
            return carry

        lax.fori_loop(0, nc, body, 0)

    return k(table, idx3)


def _sc_scatter_add(msg, dst3, zeros):
    wb = N // 10

    @functools.partial(
        pl.kernel,
        out_type=jax.ShapeDtypeStruct((NC, N, HP), jnp.float32),
        mesh=_sc_mesh(),
        scratch_types=[
            pltpu.VMEM((T, CHUNK), jnp.int32),
            pltpu.VMEM((2, CHUNK, HP), jnp.float32),
            pltpu.VMEM_SHARED((N, HP), jnp.float32),
            pltpu.SemaphoreType.DMA((2,)),
        ],
    )
    def k(msg_hbm, dst_hbm, zeros_hbm, out_hbm, dst_v, rows_v, acc_sh, sem):
        cid = lax.axis_index("c")
        sid = lax.axis_index("s")
        wid = sid * NC + cid
        nc = jnp.maximum(0, jnp.minimum(T, NJ - wid * T))

        @pl.when(sid == 0)
        def _():
            pltpu.sync_copy(zeros_hbm, acc_sh)

        plsc.subcore_barrier()
        pltpu.sync_copy(dst_hbm.at[wid], dst_v)

        @pl.when(nc > 0)
        def _():
            pltpu.async_copy(msg_hbm.at[pl.ds(wid * T * CHUNK, CHUNK)],
                             rows_v.at[0], sem.at[0])

        def body(t, carry):
            p = lax.rem(t, 2)
            pn = lax.rem(t + 1, 2)

            @pl.when(t + 1 < nc)
            def _():
                pltpu.async_copy(
                    msg_hbm.at[pl.ds((wid * T + t + 1) * CHUNK, CHUNK)],
                    rows_v.at[pn], sem.at[pn])

            pltpu.make_async_copy(
                msg_hbm.at[pl.ds((wid * T + t) * CHUNK, CHUNK)],
                rows_v.at[p], sem.at[p]).wait()
            pltpu.sync_copy(rows_v.at[p], acc_sh.at[dst_v.at[t]], add=True)
            return carry

        lax.fori_loop(0, nc, body, 0)
        plsc.subcore_barrier()

        @pl.when(sid < 10)
        def _():
            pltpu.sync_copy(acc_sh.at[pl.ds(sid * wb, wb)],
                            out_hbm.at[cid].at[pl.ds(sid * wb, wb)])

    return k(msg, dst3, zeros)


_BE = 3200
_BN = 2000


def _tc_pre_mlp(X, p0W, p0b, p1W, p1b, p2W, p2b):
    def body(x_ref, w0, b0, w1, b1, w2, b2, o_ref):
        o = jnp.maximum(jnp.dot(x_ref[...], w0[...],
                                preferred_element_type=jnp.float32) + b0[...], 0.0)
        o = jnp.maximum(jnp.dot(o, w1[...],
                                preferred_element_type=jnp.float32) + b1[...], 0.0)
        o = jnp.maximum(jnp.dot(o, w2[...],
                                preferred_element_type=jnp.float32) + b2[...], 0.0)
        o_ref[...] = jnp.concatenate(
            [o, jnp.zeros((_BN, HP - H), jnp.float32)], axis=1)

    full = lambda shape: pl.BlockSpec(shape, lambda i: (0, 0))
    return pl.pallas_call(
        body,
        grid=(N // _BN,),
        in_specs=[pl.BlockSpec((_BN, DF), lambda i: (i, 0)),
                  full((DF, H)), full((1, H)),
                  full((H, H)), full((1, H)),
                  full((H, H)), full((1, H))],
        out_specs=pl.BlockSpec((_BN, HP), lambda i: (i, 0)),
        out_shape=jax.ShapeDtypeStruct((N, HP), jnp.float32),
    )(X, p0W, p0b, p1W, p1b, p2W, p2b)


def _tc_messages(ea, y, w0, b0, W1a, Rt):
    def body(ea_ref, y_ref, w0r, b0r, W1r, Rtr, o_ref):
        u1 = jnp.maximum(ea_ref[...] * w0r[...] + b0r[...], 0.0)
        wf = jnp.dot(u1.astype(jnp.bfloat16), W1r[...],
                     preferred_element_type=jnp.float32)
        yt = jnp.tile(y_ref[...][:, :H], (1, H))
        P = (wf * yt).astype(jnp.bfloat16)
        msg = jnp.dot(P, Rtr[...], preferred_element_type=jnp.float32)
        o_ref[:, :H] = msg
        o_ref[:, H:2 * H] = jnp.ones((_BE, H), jnp.float32)

    full = lambda shape: pl.BlockSpec(shape, lambda i: (0, 0))
    return pl.pallas_call(
        body,
        grid=(E // _BE,),
        in_specs=[pl.BlockSpec((_BE, 1), lambda i: (i, 0)),
                  pl.BlockSpec((_BE, HP), lambda i: (i, 0)),
                  full((1, GH + 1)), full((1, GH + 1)),
                  full((GH + 1, HH)), full((HH, H))],
        out_specs=pl.BlockSpec((_BE, HP), lambda i: (i, 0)),
        out_shape=jax.ShapeDtypeStruct((E, HP), jnp.float32),
    )(ea, y, w0, b0, W1a, Rt)


def _tc_gru(h, agg0, agg1, Wroot, broot, WihT, bih, WhhT, bhh):
    def body(h_ref, a0, a1, wr, br, wih, bi, whh, bh, o_ref):
        hv = h_ref[...][:, :H]
        a0v = a0[...]
        a1v = a1[...]
        cnt = a0v[:, H:H + 1] + a1v[:, H:H + 1]
        inv = 1.0 / jnp.maximum(cnt, 1.0)
        agg = (a0v[:, :H] + a1v[:, :H]) * inv
        conv = jnp.dot(hv, wr[...],
                       preferred_element_type=jnp.float32) + br[...] + agg
        gi = jnp.dot(conv, wih[...],
                     preferred_element_type=jnp.float32) + bi[...]
        gh = jnp.dot(hv, whh[...],
                     preferred_element_type=jnp.float32) + bh[...]
        r = jax.nn.sigmoid(gi[:, 0:H] + gh[:, 0:H])
        z = jax.nn.sigmoid(gi[:, H:2 * H] + gh[:, H:2 * H])
        n = jnp.tanh(gi[:, 2 * H:] + r * gh[:, 2 * H:])
        o = (1.0 - z) * n + z * hv
        o_ref[...] = jnp.concatenate(
            [o, jnp.zeros((_BN, HP - H), jnp.float32)], axis=1)

    full = lambda shape: pl.BlockSpec(shape, lambda i: (0, 0))
    blk = lambda w: pl.BlockSpec((_BN, w), lambda i: (i, 0))
    return pl.pallas_call(
        body,
        grid=(N // _BN,),
        in_specs=[blk(HP), blk(HP), blk(HP),
                  full((H, H)), full((1, H)),
                  full((H, 3 * H)), full((1, 3 * H)),
                  full((H, 3 * H)), full((1, 3 * H))],
        out_specs=blk(HP),
        out_shape=jax.ShapeDtypeStruct((N, HP), jnp.float32),
    )(h, agg0, agg1, Wroot, broot, WihT, bih, WhhT, bhh)


def _tc_pool_post(x, bm, q0W, q0b, q1W, q1b, q2W, q2b, q3W, q3b):
    G = N // _BN

    def body(x_ref, bm_ref, w0, b0, w1, b1, w2, b2, w3, b3, o_ref, acc, cnt):
        i = pl.program_id(0)

        @pl.when(i == 0)
        def _():
            acc[...] = jnp.zeros_like(acc)
            cnt[...] = jnp.zeros_like(cnt)

        ids = bm_ref[...]
        onehot = (ids == lax.broadcasted_iota(jnp.int32, (_BN, NG), 1)
                  ).astype(jnp.float32)
        dn = (((0,), (0,)), ((), ()))
        acc[...] += lax.dot_general(onehot, x_ref[...][:, :H], dn,
                                    preferred_element_type=jnp.float32)
        cnt[...] += lax.dot_general(onehot, jnp.ones((_BN, H), jnp.float32), dn,
                                    preferred_element_type=jnp.float32)

        @pl.when(i == G - 1)
        def _():
            m = acc[...] / jnp.maximum(cnt[...], 1.0)
            o = jnp.maximum(jnp.dot(m, w0[...],
                                    preferred_element_type=jnp.float32) + b0[...], 0.0)
            o = jnp.maximum(jnp.dot(o, w1[...],
                                    preferred_element_type=jnp.float32) + b1[...], 0.0)
            o = jnp.maximum(jnp.dot(o, w2[...],
                                    preferred_element_type=jnp.float32) + b2[...], 0.0)
            o_ref[...] = jnp.dot(o, w3[...],
                                 preferred_element_type=jnp.float32) + b3[...]

    full = lambda shape: pl.BlockSpec(shape, lambda i: (0, 0))
    return pl.pallas_call(
        body,
        grid=(G,),
        in_specs=[pl.BlockSpec((_BN, HP), lambda i: (i, 0)),
                  pl.BlockSpec((_BN, 1), lambda i: (i, 0)),
                  full((H, H)), full((1, H)),
                  full((H, H)), full((1, H)),
                  full((H, H)), full((1, H)),
                  full((H, 1)), full((1, 1))],
        out_specs=pl.BlockSpec((NG, 1), lambda i: (0, 0)),
        out_shape=jax.ShapeDtypeStruct((NG, 1), jnp.float32),
        scratch_shapes=[pltpu.VMEM((NG, H), jnp.float32),
                        pltpu.VMEM((NG, H), jnp.float32)],
    )(x, bm, q0W, q0b, q1W, q1b, q2W, q2b, q3W, q3b)


def kernel(X, edge_idx, edge_attr, batch_map, params):
    pad = jnp.zeros((EPAD - E,), jnp.int32)
    src2 = jnp.concatenate([edge_idx[0], pad]).reshape(NW, T, CHUNK)
    dst2 = jnp.concatenate([edge_idx[1], pad]).reshape(NW, T, CHUNK)
    ea = edge_attr.reshape(E, 1)
    bm = batch_map.reshape(N, 1)
    zeros = jnp.zeros((N, HP), jnp.float32)

    row = lambda b: b.reshape(1, -1)
    Rt = jnp.kron(jnp.eye(H, dtype=jnp.bfloat16),
                  jnp.ones((H, 1), jnp.bfloat16))

    pre = params['pre']
    out = _tc_pre_mlp(X, pre[0]['W'], row(pre[0]['b']),
                      pre[1]['W'], row(pre[1]['b']),
                      pre[2]['W'], row(pre[2]['b']))

    for g in params['gcn']:
        W1 = g['edge_nn1']['W']
        b1 = g['edge_nn1']['b']
        W1p = W1.reshape(GH, H, H).transpose(0, 2, 1).reshape(GH, HH)
        b1p = b1.reshape(H, H).T.reshape(1, HH)
        W1a = jnp.concatenate([W1p, b1p], axis=0).astype(jnp.bfloat16)

        w0a = jnp.concatenate([g['edge_nn0']['W'].reshape(1, GH),
                               jnp.zeros((1, 1), jnp.float32)], axis=1)
        b0a = jnp.concatenate([g['edge_nn0']['b'].reshape(1, GH),
                               jnp.ones((1, 1), jnp.float32)], axis=1)

        y = _sc_gather(out, src2)
        msg = _tc_messages(ea, y, w0a, b0a, W1a, Rt)
        aggp = _sc_scatter_add(msg, dst2, zeros)
        out = _tc_gru(out, aggp[0], aggp[1],
                      g['root']['W'], row(g['root']['b']),
                      g['gru']['W_ih'].T, row(g['gru']['b_ih']),
                      g['gru']['W_hh'].T, row(g['gru']['b_hh']))

    post = params['post']
    return _tc_pool_post(out, bm,
                         post[0]['W'], row(post[0]['b']),
                         post[1]['W'], row(post[1]['b']),
                         post[2]['W'], row(post[2]['b']),
                         post[3]['W'], row(post[3]['b']))


# speedup vs baseline: 1.0922x; 1.0089x over previous
import functools

import jax
import jax.numpy as jnp
from jax import lax
from jax.experimental import pallas as pl
from jax.experimental.pallas import tpu as pltpu
from jax.experimental.pallas import tpu_sc as plsc

N = 10000
E = 160000
DF = 128
H = 32
HP = 128
GH = 64
NG = 128
HH = H * H

NC, NS = 2, 16
NW = NC * NS
CHUNK = 128
NJ = E // CHUNK
T = -(-NJ // NW)
EPAD = NW * T * CHUNK


def _sc_mesh():
    return plsc.VectorSubcoreMesh(core_axis_name="c", subcore_axis_name="s",
                                  num_cores=NC, num_subcores=NS)


def _sc_gather(table, idx3):
    @functools.partial(
        pl.kernel,
        out_type=jax.ShapeDtypeStruct((E, HP), jnp.float32),
        mesh=_sc_mesh(),
        scratch_types=[
            pltpu.VMEM((T, CHUNK), jnp.int32),
            pltpu.VMEM((2, CHUNK, HP), jnp.float32),
            pltpu.SemaphoreType.DMA((2,)),
        ],
    )
    def k(table_hbm, idx_hbm, out_hbm, idx_v, rows_v, sem):
        wid = lax.axis_index("s") * NC + lax.axis_index("c")
        nc = jnp.maximum(0, jnp.minimum(T, NJ - wid * T))
        pltpu.sync_copy(idx_hbm.at[wid], idx_v)

        @pl.when(nc > 0)
        def _():
            pltpu.async_copy(table_hbm.at[idx_v.at[0]], rows_v.at[0], sem.at[0])

        def body(t, carry):
            p = lax.rem(t, 2)
            pn = lax.rem(t + 1, 2)

            @pl.when(t + 1 < nc)
            def _():
                pltpu.async_copy(table_hbm.at[idx_v.at[t + 1]],
                                 rows_v.at[pn], sem.at[pn])

            pltpu.make_async_copy(table_hbm.at[idx_v.at[t]],
                                  rows_v.at[p], sem.at[p]).wait()
            pltpu.sync_copy(rows_v.at[p],
                            out_hbm.at[pl.ds((wid * T + t) * CHUNK, CHUNK)])
            return carry

        lax.fori_loop(0, nc, body, 0)

    return k(table, idx3)


def _sc_scatter_add(msg, dst3, zeros):
    wb = N // 10

    @functools.partial(
        pl.kernel,
        out_type=jax.ShapeDtypeStruct((NC, N, HP), jnp.float32),
        mesh=_sc_mesh(),
        scratch_types=[
            pltpu.VMEM((T, CHUNK), jnp.int32),
            pltpu.VMEM((2, CHUNK, HP), jnp.float32),
            pltpu.VMEM_SHARED((N, HP), jnp.float32),
            pltpu.SemaphoreType.DMA((2,)),
        ],
    )
    def k(msg_hbm, dst_hbm, zeros_hbm, out_hbm, dst_v, rows_v, acc_sh, sem):
        cid = lax.axis_index("c")
        sid = lax.axis_index("s")
        wid = sid * NC + cid
        nc = jnp.maximum(0, jnp.minimum(T, NJ - wid * T))

        @pl.when(sid == 0)
        def _():
            pltpu.sync_copy(zeros_hbm, acc_sh)

        plsc.subcore_barrier()
        pltpu.sync_copy(dst_hbm.at[wid], dst_v)

        @pl.when(nc > 0)
        def _():
            pltpu.async_copy(msg_hbm.at[pl.ds(wid * T * CHUNK, CHUNK)],
                             rows_v.at[0], sem.at[0])

        def body(t, carry):
            p = lax.rem(t, 2)
            pn = lax.rem(t + 1, 2)

            @pl.when(t + 1 < nc)
            def _():
                pltpu.async_copy(
                    msg_hbm.at[pl.ds((wid * T + t + 1) * CHUNK, CHUNK)],
                    rows_v.at[pn], sem.at[pn])

            pltpu.make_async_copy(
                msg_hbm.at[pl.ds((wid * T + t) * CHUNK, CHUNK)],
                rows_v.at[p], sem.at[p]).wait()
            pltpu.sync_copy(rows_v.at[p], acc_sh.at[dst_v.at[t]], add=True)
            return carry

        lax.fori_loop(0, nc, body, 0)
        plsc.subcore_barrier()

        @pl.when(sid < 10)
        def _():
            pltpu.sync_copy(acc_sh.at[pl.ds(sid * wb, wb)],
                            out_hbm.at[cid].at[pl.ds(sid * wb, wb)])

    return k(msg, dst3, zeros)


_BE = 4000
_BN = 2000


def _tc_pre_mlp(X, p0W, p0b, p1W, p1b, p2W, p2b):
    def body(x_ref, w0, b0, w1, b1, w2, b2, o_ref):
        o = jnp.maximum(jnp.dot(x_ref[...], w0[...],
                                preferred_element_type=jnp.float32) + b0[...], 0.0)
        o = jnp.maximum(jnp.dot(o, w1[...],
                                preferred_element_type=jnp.float32) + b1[...], 0.0)
        o = jnp.maximum(jnp.dot(o, w2[...],
                                preferred_element_type=jnp.float32) + b2[...], 0.0)
        o_ref[...] = jnp.concatenate(
            [o, jnp.zeros((_BN, HP - H), jnp.float32)], axis=1)

    full = lambda shape: pl.BlockSpec(shape, lambda i: (0, 0))
    return pl.pallas_call(
        body,
        grid=(N // _BN,),
        in_specs=[pl.BlockSpec((_BN, DF), lambda i: (i, 0)),
                  full((DF, H)), full((1, H)),
                  full((H, H)), full((1, H)),
                  full((H, H)), full((1, H))],
        out_specs=pl.BlockSpec((_BN, HP), lambda i: (i, 0)),
        out_shape=jax.ShapeDtypeStruct((N, HP), jnp.float32),
    )(X, p0W, p0b, p1W, p1b, p2W, p2b)


def _tc_messages(ea, y, w0, b0, W1a, Rt):
    def body(ea_ref, y_ref, w0r, b0r, W1r, Rtr, o_ref):
        u1 = jnp.maximum(ea_ref[...] * w0r[...] + b0r[...], 0.0)
        wf = jnp.dot(u1.astype(jnp.bfloat16), W1r[...],
                     preferred_element_type=jnp.float32)
        yt = jnp.tile(y_ref[...][:, :H], (1, H))
        P = (wf * yt).astype(jnp.bfloat16)
        msg = jnp.dot(P, Rtr[...], preferred_element_type=jnp.float32)
        o_ref[:, :H] = msg
        o_ref[:, H:2 * H] = jnp.ones((_BE, H), jnp.float32)

    full = lambda shape: pl.BlockSpec(shape, lambda i: (0, 0))
    return pl.pallas_call(
        body,
        grid=(E // _BE,),
        in_specs=[pl.BlockSpec((_BE, 1), lambda i: (i, 0)),
                  pl.BlockSpec((_BE, HP), lambda i: (i, 0)),
                  full((1, GH + 1)), full((1, GH + 1)),
                  full((GH + 1, HH)), full((HH, H))],
        out_specs=pl.BlockSpec((_BE, HP), lambda i: (i, 0)),
        out_shape=jax.ShapeDtypeStruct((E, HP), jnp.float32),
    )(ea, y, w0, b0, W1a, Rt)


def _tc_gru(h, agg0, agg1, Wroot, broot, WihT, bih, WhhT, bhh):
    def body(h_ref, a0, a1, wr, br, wih, bi, whh, bh, o_ref):
        hv = h_ref[...][:, :H]
        a0v = a0[...]
        a1v = a1[...]
        cnt = a0v[:, H:H + 1] + a1v[:, H:H + 1]
        inv = 1.0 / jnp.maximum(cnt, 1.0)
        agg = (a0v[:, :H] + a1v[:, :H]) * inv
        conv = jnp.dot(hv, wr[...],
                       preferred_element_type=jnp.float32) + br[...] + agg
        gi = jnp.dot(conv, wih[...],
                     preferred_element_type=jnp.float32) + bi[...]
        gh = jnp.dot(hv, whh[...],
                     preferred_element_type=jnp.float32) + bh[...]
        r = jax.nn.sigmoid(gi[:, 0:H] + gh[:, 0:H])
        z = jax.nn.sigmoid(gi[:, H:2 * H] + gh[:, H:2 * H])
        n = jnp.tanh(gi[:, 2 * H:] + r * gh[:, 2 * H:])
        o = (1.0 - z) * n + z * hv
        o_ref[...] = jnp.concatenate(
            [o, jnp.zeros((_BN, HP - H), jnp.float32)], axis=1)

    full = lambda shape: pl.BlockSpec(shape, lambda i: (0, 0))
    blk = lambda w: pl.BlockSpec((_BN, w), lambda i: (i, 0))
    return pl.pallas_call(
        body,
        grid=(N // _BN,),
        in_specs=[blk(HP), blk(HP), blk(HP),
                  full((H, H)), full((1, H)),
                  full((H, 3 * H)), full((1, 3 * H)),
                  full((H, 3 * H)), full((1, 3 * H))],
        out_specs=blk(HP),
        out_shape=jax.ShapeDtypeStruct((N, HP), jnp.float32),
    )(h, agg0, agg1, Wroot, broot, WihT, bih, WhhT, bhh)


def _tc_pool_post(x, bm, q0W, q0b, q1W, q1b, q2W, q2b, q3W, q3b):
    G = N // _BN

    def body(x_ref, bm_ref, w0, b0, w1, b1, w2, b2, w3, b3, o_ref, acc, cnt):
        i = pl.program_id(0)

        @pl.when(i == 0)
        def _():
            acc[...] = jnp.zeros_like(acc)
            cnt[...] = jnp.zeros_like(cnt)

        ids = bm_ref[...]
        onehot = (ids == lax.broadcasted_iota(jnp.int32, (_BN, NG), 1)
                  ).astype(jnp.float32)
        dn = (((0,), (0,)), ((), ()))
        acc[...] += lax.dot_general(onehot, x_ref[...][:, :H], dn,
                                    preferred_element_type=jnp.float32)
        cnt[...] += lax.dot_general(onehot, jnp.ones((_BN, H), jnp.float32), dn,
                                    preferred_element_type=jnp.float32)

        @pl.when(i == G - 1)
        def _():
            m = acc[...] / jnp.maximum(cnt[...], 1.0)
            o = jnp.maximum(jnp.dot(m, w0[...],
                                    preferred_element_type=jnp.float32) + b0[...], 0.0)
            o = jnp.maximum(jnp.dot(o, w1[...],
                                    preferred_element_type=jnp.float32) + b1[...], 0.0)
            o = jnp.maximum(jnp.dot(o, w2[...],
                                    preferred_element_type=jnp.float32) + b2[...], 0.0)
            o_ref[...] = jnp.dot(o, w3[...],
                                 preferred_element_type=jnp.float32) + b3[...]

    full = lambda shape: pl.BlockSpec(shape, lambda i: (0, 0))
    return pl.pallas_call(
        body,
        grid=(G,),
        in_specs=[pl.BlockSpec((_BN, HP), lambda i: (i, 0)),
                  pl.BlockSpec((_BN, 1), lambda i: (i, 0)),
                  full((H, H)), full((1, H)),
                  full((H, H)), full((1, H)),
                  full((H, H)), full((1, H)),
                  full((H, 1)), full((1, 1))],
        out_specs=pl.BlockSpec((NG, 1), lambda i: (0, 0)),
        out_shape=jax.ShapeDtypeStruct((NG, 1), jnp.float32),
        scratch_shapes=[pltpu.VMEM((NG, H), jnp.float32),
                        pltpu.VMEM((NG, H), jnp.float32)],
    )(x, bm, q0W, q0b, q1W, q1b, q2W, q2b, q3W, q3b)


def kernel(X, edge_idx, edge_attr, batch_map, params):
    pad = jnp.zeros((EPAD - E,), jnp.int32)
    src2 = jnp.concatenate([edge_idx[0], pad]).reshape(NW, T, CHUNK)
    dst2 = jnp.concatenate([edge_idx[1], pad]).reshape(NW, T, CHUNK)
    ea = edge_attr.reshape(E, 1)
    bm = batch_map.reshape(N, 1)
    zeros = jnp.zeros((N, HP), jnp.float32)

    row = lambda b: b.reshape(1, -1)
    Rt = jnp.kron(jnp.eye(H, dtype=jnp.bfloat16),
                  jnp.ones((H, 1), jnp.bfloat16))

    pre = params['pre']
    out = _tc_pre_mlp(X, pre[0]['W'], row(pre[0]['b']),
                      pre[1]['W'], row(pre[1]['b']),
                      pre[2]['W'], row(pre[2]['b']))

    for g in params['gcn']:
        W1 = g['edge_nn1']['W']
        b1 = g['edge_nn1']['b']
        W1p = W1.reshape(GH, H, H).transpose(0, 2, 1).reshape(GH, HH)
        b1p = b1.reshape(H, H).T.reshape(1, HH)
        W1a = jnp.concatenate([W1p, b1p], axis=0).astype(jnp.bfloat16)

        w0a = jnp.concatenate([g['edge_nn0']['W'].reshape(1, GH),
                               jnp.zeros((1, 1), jnp.float32)], axis=1)
        b0a = jnp.concatenate([g['edge_nn0']['b'].reshape(1, GH),
                               jnp.ones((1, 1), jnp.float32)], axis=1)

        y = _sc_gather(out, src2)
        msg = _tc_messages(ea, y, w0a, b0a, W1a, Rt)
        aggp = _sc_scatter_add(msg, dst2, zeros)
        out = _tc_gru(out, aggp[0], aggp[1],
                      g['root']['W'], row(g['root']['b']),
                      g['gru']['W_ih'].T, row(g['gru']['b_ih']),
                      g['gru']['W_hh'].T, row(g['gru']['b_hh']))

    post = params['post']
    return _tc_pool_post(out, bm,
                         post[0]['W'], row(post[0]['b']),
                         post[1]['W'], row(post[1]['b']),
                         post[2]['W'], row(post[2]['b']),
                         post[3]['W'], row(post[3]['b']))
